# Initial kernel scaffold; baseline (speedup 1.0000x reference)
import jax
import jax.numpy as jnp
from jax.experimental import pallas as pl


def kernel(x, edge_index, batch, W_l, b_l, W_r, gamma, beta, W1, b1, W2, b2, Wa, ba):
    raise NotImplementedError("write your pallas kernel here")


# speedup vs baseline: 8.9057x; 8.9057x over previous
import functools

import jax
import jax.numpy as jnp
from jax import lax
from jax.experimental import pallas as pl
from jax.experimental.pallas import tpu as pltpu
from jax.experimental.pallas import tpu_sc as plsc

N = 10000
E = 320000
D = 128
H = 32
G = 64
C = 2

NCORES = 2
NSUB = 16
NTILES = NCORES * NSUB
KC = 128
NCHUNK = 80
EPAD = NTILES * NCHUNK * KC
RPT = 640
NPAD = NSUB * RPT
DUMMY = 10100

_HIGH = lax.Precision.HIGHEST


def _proj_body(x_ref, w_ref, yl_ref, yr_ref):
    y = lax.dot_general(x_ref[...], w_ref[...], (((1,), (0,)), ((), ())),
                        preferred_element_type=jnp.float32, precision=_HIGH)
    yl_ref[...] = y[:, :H]
    yr_ref[...] = y[:, H:]


def _project(x, w_cat):
    return pl.pallas_call(
        _proj_body,
        out_shape=[jax.ShapeDtypeStruct((N, H), jnp.float32),
                   jax.ShapeDtypeStruct((N, H), jnp.float32)],
    )(x, w_cat)


def _sc_agg_body(yl_hbm, src_hbm, dst_hbm, zout_hbm, cout_hbm,
                 src_v, dst_v, rows_v, ones_v, zacc, cacc, sem):
    cid = lax.axis_index("c")
    sid = lax.axis_index("s")
    wid = cid * NSUB + sid
    rbase = sid * RPT

    zero16 = jnp.zeros((16,), jnp.float32)
    one_first = jnp.where(lax.iota(jnp.int32, 16) == 0, 1.0, 0.0)

    def _zero_rows(i, carry):
        rows_v[i, pl.ds(0, 16)] = zero16
        rows_v[i, pl.ds(16, 16)] = zero16
        ones_v[i, pl.ds(0, 16)] = zero16
        return carry

    lax.fori_loop(0, KC, _zero_rows, 0)

    for k in range(RPT // KC):
        pltpu.sync_copy(rows_v, zacc.at[pl.ds(rbase + k * KC, KC)])
        pltpu.sync_copy(ones_v, cacc.at[pl.ds(rbase + k * KC, KC)])

    def _ones_rows(i, carry):
        ones_v[i, pl.ds(0, 16)] = one_first
        return carry

    lax.fori_loop(0, KC, _ones_rows, 0)

    pltpu.sync_copy(src_hbm.at[wid], src_v)
    pltpu.sync_copy(dst_hbm.at[wid], dst_v)
    plsc.subcore_barrier()

    def _chunk(j, carry):
        pltpu.async_copy(yl_hbm.at[src_v.at[j]], rows_v, sem).wait()
        pltpu.sync_copy(rows_v, zacc.at[dst_v.at[j]], add=True)
        pltpu.sync_copy(ones_v, cacc.at[dst_v.at[j]], add=True)
        return carry

    lax.fori_loop(0, NCHUNK, _chunk, 0)
    plsc.subcore_barrier()

    for k in range(RPT // KC):
        pltpu.sync_copy(zacc.at[pl.ds(rbase + k * KC, KC)], rows_v)
        pltpu.sync_copy(rows_v, zout_hbm.at[cid, pl.ds(rbase + k * KC, KC)])
        pltpu.sync_copy(cacc.at[pl.ds(rbase + k * KC, KC)], ones_v)
        pltpu.sync_copy(ones_v, cout_hbm.at[cid, pl.ds(rbase + k * KC, KC)])


@functools.cache
def _sc_agg():
    mesh = plsc.VectorSubcoreMesh(core_axis_name="c", subcore_axis_name="s")
    return pl.kernel(
        _sc_agg_body,
        mesh=mesh,
        compiler_params=pltpu.CompilerParams(use_tc_tiling_on_sc=False),
        out_type=[jax.ShapeDtypeStruct((NCORES, NPAD, H), jnp.float32),
                  jax.ShapeDtypeStruct((NCORES, NPAD, 16), jnp.float32)],
        scratch_types=[
            pltpu.VMEM((NCHUNK, KC), jnp.int32),
            pltpu.VMEM((NCHUNK, KC), jnp.int32),
            pltpu.VMEM((KC, H), jnp.float32),
            pltpu.VMEM((KC, 16), jnp.float32),
            pltpu.VMEM_SHARED((NPAD, H), jnp.float32),
            pltpu.VMEM_SHARED((NPAD, 16), jnp.float32),
            pltpu.SemaphoreType.DMA,
        ],
    )


def _head_body(zp_ref, cp_ref, yr_ref, batch_ref, bl_ref, ga_ref, be_ref,
               w1_ref, b1_ref, w2_ref, b2_ref, wa_ref, ba_ref,
               out_ref, an_ref):
    z = (zp_ref[0] + zp_ref[1])[:N, :]
    cnt = (cp_ref[0] + cp_ref[1])[:N, 0:1]
    h = z / jnp.maximum(cnt, 1.0) + yr_ref[...] + bl_ref[...]
    mean = jnp.mean(h, axis=0, keepdims=True)
    var = jnp.mean((h - mean) ** 2, axis=0, keepdims=True)
    hn = (h - mean) * lax.rsqrt(var + 1e-5) * ga_ref[...] + be_ref[...]
    hr = jnp.maximum(hn, 0.0)

    gid = lax.broadcasted_iota(jnp.int32, (N, G), 1)
    p = (batch_ref[...] == gid).astype(jnp.float32)
    psum = lax.dot_general(p, hr, (((0,), (0,)), ((), ())),
                           preferred_element_type=jnp.float32, precision=_HIGH)
    gcnt = lax.dot_general(p, jnp.ones((N, 1), jnp.float32),
                           (((0,), (0,)), ((), ())),
                           preferred_element_type=jnp.float32, precision=_HIGH)
    pooled = psum / jnp.maximum(gcnt, 1.0)

    hid = jnp.maximum(
        lax.dot_general(pooled, w1_ref[...], (((1,), (0,)), ((), ())),
                        preferred_element_type=jnp.float32, precision=_HIGH)
        + b1_ref[...], 0.0)
    out_ref[...] = lax.dot_general(hid, w2_ref[...], (((1,), (0,)), ((), ())),
                                   preferred_element_type=jnp.float32,
                                   precision=_HIGH) + b2_ref[...]
    sa = lax.dot_general(pooled, wa_ref[...], (((1,), (0,)), ((), ())),
                         preferred_element_type=jnp.float32,
                         precision=_HIGH) + ba_ref[...]
    an_ref[...] = 1.0 / (1.0 + jnp.exp(-sa))


def _heads(zp, cp, yr, batch2, b_l, gamma, beta, W1, b1, W2, b2, Wa, ba):
    return pl.pallas_call(
        _head_body,
        out_shape=[jax.ShapeDtypeStruct((G, C), jnp.float32),
                   jax.ShapeDtypeStruct((G, 1), jnp.float32)],
    )(zp, cp, yr, batch2, b_l, gamma, beta, W1, b1, W2, b2, Wa, ba)


def kernel(x, edge_index, batch, W_l, b_l, W_r, gamma, beta,
           W1, b1, W2, b2, Wa, ba):
    w_cat = jnp.concatenate([W_l, W_r], axis=1)
    yl, yr = _project(x, w_cat)

    pad = EPAD - E
    src = jnp.concatenate([edge_index[0], jnp.zeros((pad,), jnp.int32)])
    dst = jnp.concatenate([edge_index[1], jnp.full((pad,), DUMMY, jnp.int32)])
    src = src.reshape(NTILES, NCHUNK, KC)
    dst = dst.reshape(NTILES, NCHUNK, KC)

    zp, cp = _sc_agg()(yl, src, dst)

    out, an = _heads(
        zp, cp, yr, batch.reshape(N, 1),
        b_l.reshape(1, H), gamma.reshape(1, H), beta.reshape(1, H),
        W1, b1.reshape(1, 16), W2, b2.reshape(1, C), Wa, ba.reshape(1, 1))
    return (out, an)


# speedup vs baseline: 10.4552x; 1.1740x over previous
import functools

import jax
import jax.numpy as jnp
from jax import lax
from jax.experimental import pallas as pl
from jax.experimental.pallas import tpu as pltpu
from jax.experimental.pallas import tpu_sc as plsc

N = 10000
E = 320000
D = 128
H = 32
G = 64
C = 2

W = 48
NCORES = 2
NSUB = 16
NTILES = NCORES * NSUB
KC = 128
GS = 4
NCHUNK = 80
NG = NCHUNK // GS
EPAD = NTILES * NCHUNK * KC
RPT = 640
NPAD = NSUB * RPT
DUMMY = 10100
GROWS = GS * KC

_HIGH = lax.Precision.HIGHEST


def _proj_body(x_ref, w_ref, ya_ref, yr_ref):
    y = lax.dot_general(x_ref[...], w_ref[...], (((1,), (0,)), ((), ())),
                        preferred_element_type=jnp.float32, precision=_HIGH)
    n1 = jnp.ones((N, 1), jnp.float32)
    ya_ref[...] = jnp.concatenate(
        [y[:, :H], n1, jnp.zeros((N, W - H - 1), jnp.float32)], axis=1)
    yr_ref[...] = y[:, H:]


def _project(x, w_cat):
    return pl.pallas_call(
        _proj_body,
        out_shape=[jax.ShapeDtypeStruct((N, W), jnp.float32),
                   jax.ShapeDtypeStruct((N, H), jnp.float32)],
    )(x, w_cat)


def _sc_agg_body(ya_hbm, src_hbm, dst_hbm, acc_hbm,
                 src_v, dst_v, rows_a, rows_b, acc,
                 sem_ga, sem_gb, sem_za, sem_zb):
    cid = lax.axis_index("c")
    sid = lax.axis_index("s")
    wid = cid * NSUB + sid
    rbase = sid * RPT

    def _gathers(g, buf, sem):
        return [pltpu.make_async_copy(
            ya_hbm.at[src_v.at[g * GS + b]],
            buf.at[pl.ds(b * KC, KC)], sem) for b in range(GS)]

    def _scatters(g, buf, sem):
        return [pltpu.make_async_copy(
            buf.at[pl.ds(b * KC, KC)],
            acc.at[dst_v.at[g * GS + b]], sem) for b in range(GS)]

    zero16 = jnp.zeros((16,), jnp.float32)

    def _zero_row(i, carry):
        for c in range(W // 16):
            rows_a[i, pl.ds(c * 16, 16)] = zero16
        return carry

    lax.fori_loop(0, GROWS, _zero_row, 0)
    pltpu.sync_copy(rows_a, acc.at[pl.ds(rbase, GROWS)])
    pltpu.sync_copy(rows_a.at[pl.ds(0, RPT - GROWS)],
                    acc.at[pl.ds(rbase + GROWS, RPT - GROWS)])

    pltpu.sync_copy(src_hbm.at[wid], src_v)
    pltpu.sync_copy(dst_hbm.at[wid], dst_v)
    plsc.subcore_barrier()

    for d in _gathers(0, rows_a, sem_ga):
        d.start()

    def _pair(i, carry):
        ga = 2 * i
        gb = 2 * i + 1

        @pl.when(i > 0)
        def _():
            for d in _scatters(gb - 2, rows_b, sem_zb):
                d.wait()

        for d in _gathers(gb, rows_b, sem_gb):
            d.start()
        for d in _gathers(ga, rows_a, sem_ga):
            d.wait()
        for d in _scatters(ga, rows_a, sem_za):
            d.start(add=True)
        for d in _scatters(ga, rows_a, sem_za):
            d.wait()

        @pl.when(i < NG // 2 - 1)
        def _():
            for d in _gathers(ga + 2, rows_a, sem_ga):
                d.start()

        for d in _gathers(gb, rows_b, sem_gb):
            d.wait()
        for d in _scatters(gb, rows_b, sem_zb):
            d.start(add=True)
        return carry

    lax.fori_loop(0, NG // 2, _pair, 0)
    for d in _scatters(NG - 1, rows_b, sem_zb):
        d.wait()
    plsc.subcore_barrier()

    pltpu.sync_copy(acc.at[pl.ds(rbase, GROWS)], rows_a)
    pltpu.sync_copy(rows_a, acc_hbm.at[cid, pl.ds(rbase, GROWS)])
    pltpu.sync_copy(acc.at[pl.ds(rbase + GROWS, RPT - GROWS)],
                    rows_b.at[pl.ds(0, RPT - GROWS)])
    pltpu.sync_copy(rows_b.at[pl.ds(0, RPT - GROWS)],
                    acc_hbm.at[cid, pl.ds(rbase + GROWS, RPT - GROWS)])


@functools.cache
def _sc_agg():
    mesh = plsc.VectorSubcoreMesh(core_axis_name="c", subcore_axis_name="s")
    return pl.kernel(
        _sc_agg_body,
        mesh=mesh,
        compiler_params=pltpu.CompilerParams(use_tc_tiling_on_sc=False),
        out_type=[jax.ShapeDtypeStruct((NCORES, NPAD, W), jnp.float32)],
        scratch_types=[
            pltpu.VMEM((NCHUNK, KC), jnp.int32),
            pltpu.VMEM((NCHUNK, KC), jnp.int32),
            pltpu.VMEM((GROWS, W), jnp.float32),
            pltpu.VMEM((GROWS, W), jnp.float32),
            pltpu.VMEM_SHARED((NPAD, W), jnp.float32),
            pltpu.SemaphoreType.DMA,
            pltpu.SemaphoreType.DMA,
            pltpu.SemaphoreType.DMA,
            pltpu.SemaphoreType.DMA,
        ],
    )


def _head_body(ap_ref, yr_ref, batch_ref, bl_ref, ga_ref, be_ref,
               w1_ref, b1_ref, w2_ref, b2_ref, wa_ref, ba_ref,
               out_ref, an_ref):
    a = ap_ref[0] + ap_ref[1]
    z = a[:N, :H]
    cnt = a[:N, H:H + 1]
    h = z / jnp.maximum(cnt, 1.0) + yr_ref[...] + bl_ref[...]
    mean = jnp.mean(h, axis=0, keepdims=True)
    var = jnp.mean((h - mean) ** 2, axis=0, keepdims=True)
    hn = (h - mean) * lax.rsqrt(var + 1e-5) * ga_ref[...] + be_ref[...]
    hr = jnp.maximum(hn, 0.0)

    gid = lax.broadcasted_iota(jnp.int32, (N, G), 1)
    p = (batch_ref[...] == gid).astype(jnp.float32)
    psum = lax.dot_general(p, hr, (((0,), (0,)), ((), ())),
                           preferred_element_type=jnp.float32, precision=_HIGH)
    gcnt = lax.dot_general(p, jnp.ones((N, 1), jnp.float32),
                           (((0,), (0,)), ((), ())),
                           preferred_element_type=jnp.float32, precision=_HIGH)
    pooled = psum / jnp.maximum(gcnt, 1.0)

    hid = jnp.maximum(
        lax.dot_general(pooled, w1_ref[...], (((1,), (0,)), ((), ())),
                        preferred_element_type=jnp.float32, precision=_HIGH)
        + b1_ref[...], 0.0)
    out_ref[...] = lax.dot_general(hid, w2_ref[...], (((1,), (0,)), ((), ())),
                                   preferred_element_type=jnp.float32,
                                   precision=_HIGH) + b2_ref[...]
    sa = lax.dot_general(pooled, wa_ref[...], (((1,), (0,)), ((), ())),
                         preferred_element_type=jnp.float32,
                         precision=_HIGH) + ba_ref[...]
    an_ref[...] = 1.0 / (1.0 + jnp.exp(-sa))


def _heads(ap, yr, batch2, b_l, gamma, beta, W1, b1, W2, b2, Wa, ba):
    return pl.pallas_call(
        _head_body,
        out_shape=[jax.ShapeDtypeStruct((G, C), jnp.float32),
                   jax.ShapeDtypeStruct((G, 1), jnp.float32)],
    )(ap, yr, batch2, b_l, gamma, beta, W1, b1, W2, b2, Wa, ba)


def kernel(x, edge_index, batch, W_l, b_l, W_r, gamma, beta,
           W1, b1, W2, b2, Wa, ba):
    w_cat = jnp.concatenate([W_l, W_r], axis=1)
    ya, yr = _project(x, w_cat)

    pad = EPAD - E
    src = jnp.concatenate([edge_index[0], jnp.zeros((pad,), jnp.int32)])
    dst = jnp.concatenate([edge_index[1], jnp.full((pad,), DUMMY, jnp.int32)])
    src = src.reshape(NTILES, NCHUNK, KC)
    dst = dst.reshape(NTILES, NCHUNK, KC)

    (ap,) = _sc_agg()(ya, src, dst)

    out, an = _heads(
        ap, yr, batch.reshape(N, 1),
        b_l.reshape(1, H), gamma.reshape(1, H), beta.reshape(1, H),
        W1, b1.reshape(1, 16), W2, b2.reshape(1, C), Wa, ba.reshape(1, 1))
    return (out, an)


# speedup vs baseline: 19.2891x; 1.8449x over previous
import functools

import jax
import jax.numpy as jnp
from jax import lax
from jax.experimental import pallas as pl
from jax.experimental.pallas import tpu as pltpu
from jax.experimental.pallas import tpu_sc as plsc

N = 10000
E = 320000
D = 128
H = 32
G = 64
C = 2

W = 48
NCORES = 2
NSUB = 16
NTILES = NCORES * NSUB
KC = 128
GS = 3
NCHUNKS_TOT = E // KC
NCHUNK = NCHUNKS_TOT // NTILES
NEXTRA = NCHUNKS_TOT - NCHUNK * NTILES
NG = NCHUNK // GS
NG2 = NG // 2
RPT = 640
NPAD = NSUB * RPT
GROWS = GS * KC

_HIGH = lax.Precision.HIGHEST


def _proj_body(x_ref, w_ref, ya_ref, yr_ref):
    y = lax.dot_general(x_ref[...], w_ref[...], (((1,), (0,)), ((), ())),
                        preferred_element_type=jnp.float32, precision=_HIGH)
    n1 = jnp.ones((N, 1), jnp.float32)
    ya_ref[...] = jnp.concatenate(
        [y[:, :H], n1, jnp.zeros((N, W - H - 1), jnp.float32)], axis=1)
    yr_ref[...] = y[:, H:]


def _project(x, w_cat):
    return pl.pallas_call(
        _proj_body,
        out_shape=[jax.ShapeDtypeStruct((N, W), jnp.float32),
                   jax.ShapeDtypeStruct((N, H), jnp.float32)],
    )(x, w_cat)


def _sc_agg_body(ya_hbm, src_hbm, dst_hbm, acc_hbm,
                 src_v, dst_v, rows_a, rows_b, acc,
                 sem_ga, sem_gb, sem_za, sem_zb):
    cid = lax.axis_index("c")
    sid = lax.axis_index("s")
    wid = cid * NSUB + sid
    rbase = sid * RPT
    cbase = wid * NCHUNK

    def _gathers(g, buf, sem):
        return [pltpu.make_async_copy(
            ya_hbm.at[src_v.at[g * GS + b]],
            buf.at[pl.ds(b * KC, KC)], sem) for b in range(GS)]

    def _scatters(g, buf, sem):
        return [pltpu.make_async_copy(
            buf.at[pl.ds(b * KC, KC)],
            acc.at[dst_v.at[g * GS + b]], sem) for b in range(GS)]

    zero16 = jnp.zeros((16,), jnp.float32)

    def _zero_row(i, carry):
        for c in range(W // 16):
            rows_a[i, pl.ds(c * 16, 16)] = zero16
        return carry

    lax.fori_loop(0, GROWS, _zero_row, 0)
    pltpu.sync_copy(rows_a, acc.at[pl.ds(rbase, GROWS)])
    pltpu.sync_copy(rows_a.at[pl.ds(0, RPT - GROWS)],
                    acc.at[pl.ds(rbase + GROWS, RPT - GROWS)])

    pltpu.sync_copy(src_hbm.at[pl.ds(cbase, NCHUNK)],
                    src_v.at[pl.ds(0, NCHUNK)])
    pltpu.sync_copy(dst_hbm.at[pl.ds(cbase, NCHUNK)],
                    dst_v.at[pl.ds(0, NCHUNK)])

    @pl.when(wid < NEXTRA)
    def _():
        pltpu.sync_copy(src_hbm.at[pl.ds(NTILES * NCHUNK + wid, 1)],
                        src_v.at[pl.ds(NCHUNK, 1)])
        pltpu.sync_copy(dst_hbm.at[pl.ds(NTILES * NCHUNK + wid, 1)],
                        dst_v.at[pl.ds(NCHUNK, 1)])
    plsc.subcore_barrier()

    for d in _gathers(0, rows_a, sem_ga):
        d.start()

    def _pair(i, carry):
        ga = 2 * i
        gb = 2 * i + 1

        @pl.when(i > 0)
        def _():
            for d in _scatters(gb - 2, rows_b, sem_zb):
                d.wait()

        for d in _gathers(gb, rows_b, sem_gb):
            d.start()
        for d in _gathers(ga, rows_a, sem_ga):
            d.wait()
        for d in _scatters(ga, rows_a, sem_za):
            d.start(add=True)
        for d in _scatters(ga, rows_a, sem_za):
            d.wait()

        @pl.when(i < NG2 - 1)
        def _():
            for d in _gathers(ga + 2, rows_a, sem_ga):
                d.start()

        for d in _gathers(gb, rows_b, sem_gb):
            d.wait()
        for d in _scatters(gb, rows_b, sem_zb):
            d.start(add=True)
        return carry

    lax.fori_loop(0, NG2, _pair, 0)
    for d in _scatters(NG - 1, rows_b, sem_zb):
        d.wait()

    @pl.when(wid < NEXTRA)
    def _():
        pltpu.async_copy(ya_hbm.at[src_v.at[NCHUNK]],
                         rows_a.at[pl.ds(0, KC)], sem_ga).wait()
        d = pltpu.make_async_copy(rows_a.at[pl.ds(0, KC)],
                                  acc.at[dst_v.at[NCHUNK]], sem_za)
        d.start(add=True)
        d.wait()

    plsc.subcore_barrier()

    pltpu.sync_copy(acc.at[pl.ds(rbase, GROWS)], rows_a)
    pltpu.sync_copy(rows_a, acc_hbm.at[cid, pl.ds(rbase, GROWS)])
    pltpu.sync_copy(acc.at[pl.ds(rbase + GROWS, RPT - GROWS)],
                    rows_b.at[pl.ds(0, RPT - GROWS)])
    pltpu.sync_copy(rows_b.at[pl.ds(0, RPT - GROWS)],
                    acc_hbm.at[cid, pl.ds(rbase + GROWS, RPT - GROWS)])


@functools.cache
def _sc_agg():
    mesh = plsc.VectorSubcoreMesh(core_axis_name="c", subcore_axis_name="s")
    return pl.kernel(
        _sc_agg_body,
        mesh=mesh,
        compiler_params=pltpu.CompilerParams(use_tc_tiling_on_sc=False),
        out_type=[jax.ShapeDtypeStruct((NCORES, NPAD, W), jnp.float32)],
        scratch_types=[
            pltpu.VMEM((NCHUNK + 1, KC), jnp.int32),
            pltpu.VMEM((NCHUNK + 1, KC), jnp.int32),
            pltpu.VMEM((GROWS, W), jnp.float32),
            pltpu.VMEM((GROWS, W), jnp.float32),
            pltpu.VMEM_SHARED((NPAD, W), jnp.float32),
            pltpu.SemaphoreType.DMA,
            pltpu.SemaphoreType.DMA,
            pltpu.SemaphoreType.DMA,
            pltpu.SemaphoreType.DMA,
        ],
    )


def _head_body(ap_ref, yr_ref, batch_ref, bl_ref, ga_ref, be_ref,
               w1_ref, b1_ref, w2_ref, b2_ref, wa_ref, ba_ref,
               out_ref, an_ref):
    a = ap_ref[0] + ap_ref[1]
    z = a[:N, :H]
    cnt = a[:N, H:H + 1]
    h = z / jnp.maximum(cnt, 1.0) + yr_ref[...] + bl_ref[...]
    mean = jnp.mean(h, axis=0, keepdims=True)
    var = jnp.mean((h - mean) ** 2, axis=0, keepdims=True)
    hn = (h - mean) * lax.rsqrt(var + 1e-5) * ga_ref[...] + be_ref[...]
    hr = jnp.maximum(hn, 0.0)

    gid = lax.broadcasted_iota(jnp.int32, (N, G), 1)
    p = (batch_ref[...] == gid).astype(jnp.float32)
    psum = lax.dot_general(p, hr, (((0,), (0,)), ((), ())),
                           preferred_element_type=jnp.float32, precision=_HIGH)
    gcnt = lax.dot_general(p, jnp.ones((N, 1), jnp.float32),
                           (((0,), (0,)), ((), ())),
                           preferred_element_type=jnp.float32, precision=_HIGH)
    pooled = psum / jnp.maximum(gcnt, 1.0)

    hid = jnp.maximum(
        lax.dot_general(pooled, w1_ref[...], (((1,), (0,)), ((), ())),
                        preferred_element_type=jnp.float32, precision=_HIGH)
        + b1_ref[...], 0.0)
    out_ref[...] = lax.dot_general(hid, w2_ref[...], (((1,), (0,)), ((), ())),
                                   preferred_element_type=jnp.float32,
                                   precision=_HIGH) + b2_ref[...]
    sa = lax.dot_general(pooled, wa_ref[...], (((1,), (0,)), ((), ())),
                         preferred_element_type=jnp.float32,
                         precision=_HIGH) + ba_ref[...]
    an_ref[...] = 1.0 / (1.0 + jnp.exp(-sa))


def _heads(ap, yr, batch2, b_l, gamma, beta, W1, b1, W2, b2, Wa, ba):
    return pl.pallas_call(
        _head_body,
        out_shape=[jax.ShapeDtypeStruct((G, C), jnp.float32),
                   jax.ShapeDtypeStruct((G, 1), jnp.float32)],
    )(ap, yr, batch2, b_l, gamma, beta, W1, b1, W2, b2, Wa, ba)


def kernel(x, edge_index, batch, W_l, b_l, W_r, gamma, beta,
           W1, b1, W2, b2, Wa, ba):
    w_cat = jnp.concatenate([W_l, W_r], axis=1)
    ya, yr = _project(x, w_cat)

    src = edge_index[0].reshape(NCHUNKS_TOT, KC)
    dst = edge_index[1].reshape(NCHUNKS_TOT, KC)

    (ap,) = _sc_agg()(ya, src, dst)

    out, an = _heads(
        ap, yr, batch.reshape(N, 1),
        b_l.reshape(1, H), gamma.reshape(1, H), beta.reshape(1, H),
        W1, b1.reshape(1, 16), W2, b2.reshape(1, C), Wa, ba.reshape(1, 1))
    return (out, an)


# speedup vs baseline: 21.5250x; 1.1159x over previous
import functools

import jax
import jax.numpy as jnp
from jax import lax
from jax.experimental import pallas as pl
from jax.experimental.pallas import tpu as pltpu
from jax.experimental.pallas import tpu_sc as plsc

N = 10000
E = 320000
D = 128
H = 32
G = 64
C = 2

W = 48
NCORES = 2
NSUB = 16
NTILES = NCORES * NSUB
KC = 128
GS = 3
NCHUNKS_TOT = E // KC
NCHUNK = NCHUNKS_TOT // NTILES
NEXTRA = NCHUNKS_TOT - NCHUNK * NTILES
NG = NCHUNK // GS
NG2 = NG // 2
RPT = 640
NPAD = NSUB * RPT
GROWS = GS * KC

_HIGH = lax.Precision.HIGHEST


def _proj_body(x_ref, w_ref, y_ref):
    y = lax.dot_general(x_ref[...], w_ref[...], (((1,), (0,)), ((), ())),
                        preferred_element_type=jnp.float32, precision=_HIGH)
    n1 = jnp.ones((N, 1), jnp.float32)
    y_ref[...] = jnp.concatenate(
        [y[:, :H], n1, jnp.zeros((N, W - H - 1), jnp.float32),
         y[:, H:], jnp.zeros((N, D - W - H), jnp.float32)], axis=1)


def _project(x, w_cat):
    return pl.pallas_call(
        _proj_body,
        out_shape=[jax.ShapeDtypeStruct((N, D), jnp.float32)],
    )(x, w_cat)


def _sc_agg_body(ya_hbm, ei_hbm, acc_hbm,
                 src_v, dst_v, rows_a, rows_b, acc,
                 sem_ga, sem_gb, sem_za, sem_zb):
    cid = lax.axis_index("c")
    sid = lax.axis_index("s")
    wid = cid * NSUB + sid
    rbase = sid * RPT
    cbase = wid * NCHUNK

    def _gathers(g, buf, sem):
        return [pltpu.make_async_copy(
            ya_hbm.at[src_v.at[g * GS + b]],
            buf.at[pl.ds(b * KC, KC)], sem) for b in range(GS)]

    def _scatters(g, buf, sem):
        return [pltpu.make_async_copy(
            buf.at[pl.ds(b * KC, KC)],
            acc.at[dst_v.at[g * GS + b]], sem) for b in range(GS)]

    zero16 = jnp.zeros((16,), jnp.float32)

    def _zero_row(i, carry):
        for c in range(W // 16):
            rows_a[i, pl.ds(c * 16, 16)] = zero16
        return carry

    lax.fori_loop(0, GROWS, _zero_row, 0)
    pltpu.sync_copy(rows_a, acc.at[pl.ds(rbase, GROWS)])
    pltpu.sync_copy(rows_a.at[pl.ds(0, RPT - GROWS)],
                    acc.at[pl.ds(rbase + GROWS, RPT - GROWS)])

    pltpu.sync_copy(ei_hbm.at[0, pl.ds(cbase, NCHUNK)],
                    src_v.at[pl.ds(0, NCHUNK)])
    pltpu.sync_copy(ei_hbm.at[1, pl.ds(cbase, NCHUNK)],
                    dst_v.at[pl.ds(0, NCHUNK)])

    @pl.when(wid < NEXTRA)
    def _():
        pltpu.sync_copy(ei_hbm.at[0, pl.ds(NTILES * NCHUNK + wid, 1)],
                        src_v.at[pl.ds(NCHUNK, 1)])
        pltpu.sync_copy(ei_hbm.at[1, pl.ds(NTILES * NCHUNK + wid, 1)],
                        dst_v.at[pl.ds(NCHUNK, 1)])
    plsc.subcore_barrier()

    for d in _gathers(0, rows_a, sem_ga):
        d.start()

    def _pair(i, carry):
        ga = 2 * i
        gb = 2 * i + 1

        @pl.when(i > 0)
        def _():
            for d in _scatters(gb - 2, rows_b, sem_zb):
                d.wait()

        for d in _gathers(gb, rows_b, sem_gb):
            d.start()
        for d in _gathers(ga, rows_a, sem_ga):
            d.wait()
        for d in _scatters(ga, rows_a, sem_za):
            d.start(add=True)
        for d in _scatters(ga, rows_a, sem_za):
            d.wait()

        @pl.when(i < NG2 - 1)
        def _():
            for d in _gathers(ga + 2, rows_a, sem_ga):
                d.start()

        for d in _gathers(gb, rows_b, sem_gb):
            d.wait()
        for d in _scatters(gb, rows_b, sem_zb):
            d.start(add=True)
        return carry

    lax.fori_loop(0, NG2, _pair, 0)
    for d in _scatters(NG - 1, rows_b, sem_zb):
        d.wait()

    @pl.when(wid < NEXTRA)
    def _():
        pltpu.async_copy(ya_hbm.at[src_v.at[NCHUNK]],
                         rows_a.at[pl.ds(0, KC)], sem_ga).wait()
        d = pltpu.make_async_copy(rows_a.at[pl.ds(0, KC)],
                                  acc.at[dst_v.at[NCHUNK]], sem_za)
        d.start(add=True)
        d.wait()

    plsc.subcore_barrier()

    pltpu.sync_copy(acc.at[pl.ds(rbase, GROWS)], rows_a)
    pltpu.sync_copy(rows_a, acc_hbm.at[cid, pl.ds(rbase, GROWS)])
    pltpu.sync_copy(acc.at[pl.ds(rbase + GROWS, RPT - GROWS)],
                    rows_b.at[pl.ds(0, RPT - GROWS)])
    pltpu.sync_copy(rows_b.at[pl.ds(0, RPT - GROWS)],
                    acc_hbm.at[cid, pl.ds(rbase + GROWS, RPT - GROWS)])


@functools.cache
def _sc_agg():
    mesh = plsc.VectorSubcoreMesh(core_axis_name="c", subcore_axis_name="s")
    return pl.kernel(
        _sc_agg_body,
        mesh=mesh,
        compiler_params=pltpu.CompilerParams(use_tc_tiling_on_sc=False),
        out_type=[jax.ShapeDtypeStruct((NCORES, NPAD, W), jnp.float32)],
        scratch_types=[
            pltpu.VMEM((NCHUNK + 1, KC), jnp.int32),
            pltpu.VMEM((NCHUNK + 1, KC), jnp.int32),
            pltpu.VMEM((GROWS, W), jnp.float32),
            pltpu.VMEM((GROWS, W), jnp.float32),
            pltpu.VMEM_SHARED((NPAD, W), jnp.float32),
            pltpu.SemaphoreType.DMA,
            pltpu.SemaphoreType.DMA,
            pltpu.SemaphoreType.DMA,
            pltpu.SemaphoreType.DMA,
        ],
    )


def _head_body(ap_ref, y_ref, batch_ref, bl_ref, ga_ref, be_ref,
               w1_ref, b1_ref, w2_ref, b2_ref, wa_ref, ba_ref,
               out_ref, an_ref):
    a = ap_ref[0] + ap_ref[1]
    z = a[:N, :H]
    cnt = a[:N, H:H + 1]
    h = z / jnp.maximum(cnt, 1.0) + y_ref[:, W:W + H] + bl_ref[...]
    mean = jnp.mean(h, axis=0, keepdims=True)
    var = jnp.mean((h - mean) ** 2, axis=0, keepdims=True)
    hn = (h - mean) * lax.rsqrt(var + 1e-5) * ga_ref[...] + be_ref[...]
    hr = jnp.maximum(hn, 0.0)

    gid = lax.broadcasted_iota(jnp.int32, (N, G), 1)
    p = (batch_ref[...] == gid).astype(jnp.float32)
    psum = lax.dot_general(p, hr, (((0,), (0,)), ((), ())),
                           preferred_element_type=jnp.float32)
    gcnt = lax.dot_general(p, jnp.ones((N, 1), jnp.float32),
                           (((0,), (0,)), ((), ())),
                           preferred_element_type=jnp.float32)
    pooled = psum / jnp.maximum(gcnt, 1.0)

    hid = jnp.maximum(
        lax.dot_general(pooled, w1_ref[...], (((1,), (0,)), ((), ())),
                        preferred_element_type=jnp.float32, precision=_HIGH)
        + b1_ref[...], 0.0)
    out_ref[...] = lax.dot_general(hid, w2_ref[...], (((1,), (0,)), ((), ())),
                                   preferred_element_type=jnp.float32,
                                   precision=_HIGH) + b2_ref[...]
    sa = lax.dot_general(pooled, wa_ref[...], (((1,), (0,)), ((), ())),
                         preferred_element_type=jnp.float32,
                         precision=_HIGH) + ba_ref[...]
    an_ref[...] = 1.0 / (1.0 + jnp.exp(-sa))


def _heads(ap, y128, batch2, b_l, gamma, beta, W1, b1, W2, b2, Wa, ba):
    return pl.pallas_call(
        _head_body,
        out_shape=[jax.ShapeDtypeStruct((G, C), jnp.float32),
                   jax.ShapeDtypeStruct((G, 1), jnp.float32)],
    )(ap, y128, batch2, b_l, gamma, beta, W1, b1, W2, b2, Wa, ba)


def kernel(x, edge_index, batch, W_l, b_l, W_r, gamma, beta,
           W1, b1, W2, b2, Wa, ba):
    w_cat = jnp.concatenate([W_l, W_r], axis=1)
    (y128,) = _project(x, w_cat)
    ya = y128[:, :W]

    ei = edge_index.reshape(2, NCHUNKS_TOT, KC)

    (ap,) = _sc_agg()(ya, ei)

    out, an = _heads(
        ap, y128, batch.reshape(N, 1),
        b_l.reshape(1, H), gamma.reshape(1, H), beta.reshape(1, H),
        W1, b1.reshape(1, 16), W2, b2.reshape(1, C), Wa, ba.reshape(1, 1))
    return (out, an)
